# Initial kernel scaffold; baseline (speedup 1.0000x reference)
import jax
import jax.numpy as jnp
from jax.experimental import pallas as pl


def kernel(h, edge_index, W, b):
    raise NotImplementedError("write your pallas kernel here")


# speedup vs baseline: 7.8090x; 7.8090x over previous
import dataclasses
import functools

import jax
import jax.numpy as jnp
from jax import lax
from jax.experimental import pallas as pl
from jax.experimental.pallas import tpu as pltpu
from jax.experimental.pallas import tpu_sc as plsc

N_NODES = 10000
N_EDGES = 320000
D = 128
NC = 2
NS = 16
L = 16
CHUNK = 128
J = 79
E_PAD = NC * NS * J * CHUNK
N_PAD = 10240
H_ROWS = N_PAD // D
A_ROWS = 10368
ROWS_PER_SUB = A_ROWS // NS

_mesh = plsc.VectorSubcoreMesh(core_axis_name="c", subcore_axis_name="s")

_cp = pltpu.CompilerParams()
if "needs_layout_passes" in pltpu.CompilerParams.__dataclass_fields__:
    _cp = dataclasses.replace(_cp, needs_layout_passes=False)


@functools.partial(
    pl.kernel,
    compiler_params=_cp,
    out_type=jax.ShapeDtypeStruct((NC, A_ROWS, D), jnp.float32),
    mesh=_mesh,
    scratch_types=[
        pltpu.VMEM((CHUNK, D), jnp.float32),
        pltpu.VMEM((H_ROWS, D), jnp.float32),
        pltpu.VMEM((CHUNK,), jnp.int32),
        pltpu.VMEM((CHUNK,), jnp.int32),
        pltpu.VMEM((H_ROWS,), jnp.int32),
        pltpu.VMEM_SHARED((A_ROWS, D), jnp.float32),
    ],
)
def _sc_aggregate(src_hbm, dst_hbm, h_hbm, p_out,
                  rows_v, hist_v, src_v, dst_v, fold_v, acc_s):
    c = lax.axis_index("c")
    s = lax.axis_index("s")

    @pl.loop(0, CHUNK)
    def _zero_rows(i):
        @pl.loop(0, D, step=L)
        def _zero_cols(jc):
            rows_v[i, pl.ds(jc, L)] = jnp.zeros((L,), jnp.float32)

    @pl.loop(0, H_ROWS)
    def _zero_hist(i):
        @pl.loop(0, D, step=L)
        def _zero_hcols(jc):
            hist_v[i, pl.ds(jc, L)] = jnp.zeros((L,), jnp.float32)
    base_row = s * ROWS_PER_SUB
    for off_k, len_k in ((0, 128), (128, 128), (256, 128), (384, 128),
                         (512, 128), (640, 8)):
        pltpu.sync_copy(rows_v.at[pl.ds(0, len_k)],
                        acc_s.at[pl.ds(base_row + off_k, len_k)])

    plsc.subcore_barrier()

    wid = c * NS + s
    lane = lax.iota(jnp.int32, L)
    ones16 = jnp.ones((L,), jnp.float32)

    @pl.loop(0, J)
    def _edge_block(j):
        off = (wid * J + j) * CHUNK
        pltpu.sync_copy(src_hbm.at[pl.ds(off, CHUNK)], src_v)
        pltpu.sync_copy(dst_hbm.at[pl.ds(off, CHUNK)], dst_v)
        pltpu.sync_copy(h_hbm.at[src_v], rows_v)
        pltpu.sync_copy(rows_v, acc_s.at[dst_v], add=True)
        for m in range(CHUNK // L):
            d16 = dst_v[pl.ds(m * L, L)]
            r16 = lax.shift_right_logical(d16, 7)
            c16 = jnp.bitwise_and(d16, 127)
            plsc.addupdate_scatter(hist_v, [r16, c16], ones16)

    @pl.loop(0, H_ROWS, step=L)
    def _fill_fold(i):
        fold_v[pl.ds(i, L)] = lane + (N_PAD + i)

    pltpu.sync_copy(hist_v, acc_s.at[fold_v], add=True)

    plsc.subcore_barrier()

    pltpu.sync_copy(acc_s.at[pl.ds(base_row, ROWS_PER_SUB)],
                    p_out.at[c].at[pl.ds(base_row, ROWS_PER_SUB)])


_BLK = 2000


def _tc_body(p_ref, d_ref, w_ref, b_ref, o_ref):
    p = p_ref[0] + p_ref[1]
    pn = p / jnp.maximum(d_ref[...], 1.0)
    acc = lax.dot_general(pn, w_ref[...], (((1,), (1,)), ((), ())),
                          preferred_element_type=jnp.float32)
    o_ref[...] = acc + b_ref[...]


def _tc_finish(p_parts, degcol, W, b2):
    return pl.pallas_call(
        _tc_body,
        grid=(N_NODES // _BLK,),
        in_specs=[
            pl.BlockSpec((NC, _BLK, D), lambda i: (0, i, 0)),
            pl.BlockSpec((_BLK, 1), lambda i: (i, 0)),
            pl.BlockSpec((D, D), lambda i: (0, 0)),
            pl.BlockSpec((1, D), lambda i: (0, 0)),
        ],
        out_specs=pl.BlockSpec((_BLK, D), lambda i: (i, 0)),
        out_shape=jax.ShapeDtypeStruct((N_NODES, D), jnp.float32),
    )(p_parts, degcol, W, b2)


def kernel(h, edge_index, W, b):
    src = edge_index[0].astype(jnp.int32)
    dst = edge_index[1].astype(jnp.int32)
    pad = E_PAD - N_EDGES
    pad_iota = jnp.arange(pad, dtype=jnp.int32)
    src_p = jnp.concatenate([src, pad_iota % N_NODES])
    dst_p = jnp.concatenate([dst, N_NODES + (pad_iota % (N_PAD - N_NODES))])
    p_parts = _sc_aggregate(src_p, dst_p, h)
    deg = p_parts[0, N_PAD:N_PAD + H_ROWS] + p_parts[1, N_PAD:N_PAD + H_ROWS]
    degcol = deg.reshape(N_PAD)[:N_NODES, None]
    return _tc_finish(p_parts, degcol, W, b.reshape(1, D))


# speedup vs baseline: 9.7924x; 1.2540x over previous
import dataclasses
import functools

import jax
import jax.numpy as jnp
from jax import lax
from jax.experimental import pallas as pl
from jax.experimental.pallas import tpu as pltpu
from jax.experimental.pallas import tpu_sc as plsc

N_NODES = 10000
N_EDGES = 320000
D = 128
NC = 2
NS = 16
L = 16
CHUNK = 128
J = 80
SUP = 8
E_PAD = NC * NS * J * CHUNK
N_PAD = 10240
H_ROWS = N_PAD // D
A_ROWS = 10368
ROWS_PER_SUB = A_ROWS // NS

_mesh = plsc.VectorSubcoreMesh(core_axis_name="c", subcore_axis_name="s")

_cp = pltpu.CompilerParams()
if "needs_layout_passes" in pltpu.CompilerParams.__dataclass_fields__:
    _cp = dataclasses.replace(_cp, needs_layout_passes=False)


@functools.partial(
    pl.kernel,
    compiler_params=_cp,
    out_type=jax.ShapeDtypeStruct((NC, A_ROWS, D), jnp.float32),
    mesh=_mesh,
    scratch_types=[
        pltpu.VMEM((CHUNK, D), jnp.float32),
        pltpu.VMEM((H_ROWS, D), jnp.float32),
        pltpu.VMEM((SUP, CHUNK), jnp.int32),
        pltpu.VMEM((SUP, CHUNK), jnp.int32),
        pltpu.VMEM((H_ROWS,), jnp.int32),
        pltpu.VMEM_SHARED((A_ROWS, D), jnp.float32),
    ],
)
def _sc_aggregate(src_hbm, dst_hbm, h_hbm, p_out,
                  rows_v, hist_v, src_v, dst_v, fold_v, acc_s):
    c = lax.axis_index("c")
    s = lax.axis_index("s")

    @pl.loop(0, CHUNK)
    def _zero_rows(i):
        @pl.loop(0, D, step=L)
        def _zero_cols(jc):
            rows_v[i, pl.ds(jc, L)] = jnp.zeros((L,), jnp.float32)

    @pl.loop(0, H_ROWS)
    def _zero_hist(i):
        @pl.loop(0, D, step=L)
        def _zero_hcols(jc):
            hist_v[i, pl.ds(jc, L)] = jnp.zeros((L,), jnp.float32)
    base_row = s * ROWS_PER_SUB
    for off_k, len_k in ((0, 128), (128, 128), (256, 128), (384, 128),
                         (512, 128), (640, 8)):
        pltpu.sync_copy(rows_v.at[pl.ds(0, len_k)],
                        acc_s.at[pl.ds(base_row + off_k, len_k)])

    plsc.subcore_barrier()

    wid = c * NS + s
    lane = lax.iota(jnp.int32, L)
    ones16 = jnp.ones((L,), jnp.float32)

    @pl.loop(0, J // SUP)
    def _super_block(sup):
        row0 = wid * J + sup * SUP
        pltpu.sync_copy(src_hbm.at[pl.ds(row0, SUP)], src_v)
        pltpu.sync_copy(dst_hbm.at[pl.ds(row0, SUP)], dst_v)

        @pl.loop(0, SUP)
        def _edge_block(jj):
            pltpu.sync_copy(h_hbm.at[src_v.at[jj]], rows_v)
            pltpu.sync_copy(rows_v, acc_s.at[dst_v.at[jj]], add=True)
            for m in range(CHUNK // L):
                d16 = dst_v[jj, pl.ds(m * L, L)]
                r16 = lax.shift_right_logical(d16, 7)
                c16 = jnp.bitwise_and(d16, 127)
                plsc.addupdate_scatter(hist_v, [r16, c16], ones16)

    @pl.loop(0, H_ROWS, step=L)
    def _fill_fold(i):
        fold_v[pl.ds(i, L)] = lane + (N_PAD + i)

    pltpu.sync_copy(hist_v, acc_s.at[fold_v], add=True)

    plsc.subcore_barrier()

    pltpu.sync_copy(acc_s.at[pl.ds(base_row, ROWS_PER_SUB)],
                    p_out.at[c].at[pl.ds(base_row, ROWS_PER_SUB)])


_BLK = 2000


def _tc_body(p_ref, d_ref, w_ref, b_ref, o_ref):
    p = p_ref[0] + p_ref[1]
    pn = p / jnp.maximum(d_ref[...], 1.0)
    acc = lax.dot_general(pn, w_ref[...], (((1,), (1,)), ((), ())),
                          preferred_element_type=jnp.float32)
    o_ref[...] = acc + b_ref[...]


def _tc_finish(p_parts, degcol, W, b2):
    return pl.pallas_call(
        _tc_body,
        grid=(N_NODES // _BLK,),
        in_specs=[
            pl.BlockSpec((NC, _BLK, D), lambda i: (0, i, 0)),
            pl.BlockSpec((_BLK, 1), lambda i: (i, 0)),
            pl.BlockSpec((D, D), lambda i: (0, 0)),
            pl.BlockSpec((1, D), lambda i: (0, 0)),
        ],
        out_specs=pl.BlockSpec((_BLK, D), lambda i: (i, 0)),
        out_shape=jax.ShapeDtypeStruct((N_NODES, D), jnp.float32),
    )(p_parts, degcol, W, b2)


def kernel(h, edge_index, W, b):
    src = edge_index[0].astype(jnp.int32)
    dst = edge_index[1].astype(jnp.int32)
    pad = E_PAD - N_EDGES
    pad_iota = jnp.arange(pad, dtype=jnp.int32)
    src_p = jnp.concatenate([src, pad_iota % N_NODES]).reshape(-1, CHUNK)
    dst_p = jnp.concatenate(
        [dst, N_NODES + (pad_iota % (N_PAD - N_NODES))]).reshape(-1, CHUNK)
    p_parts = _sc_aggregate(src_p, dst_p, h)
    deg = p_parts[0, N_PAD:N_PAD + H_ROWS] + p_parts[1, N_PAD:N_PAD + H_ROWS]
    degcol = deg.reshape(N_PAD)[:N_NODES, None]
    return _tc_finish(p_parts, degcol, W, b.reshape(1, D))


# speedup vs baseline: 12.1178x; 1.2375x over previous
import dataclasses
import functools

import jax
import jax.numpy as jnp
from jax import lax
from jax.experimental import pallas as pl
from jax.experimental.pallas import tpu as pltpu
from jax.experimental.pallas import tpu_sc as plsc

N_NODES = 10000
N_EDGES = 320000
D = 128
NC = 2
NS = 16
L = 16
CHUNK = 128
J = 80
SUP = 8
E_PAD = NC * NS * J * CHUNK
N_PAD = 10240
H_ROWS = N_PAD // D
A_ROWS = 10368
ROWS_PER_SUB = A_ROWS // NS

_mesh = plsc.VectorSubcoreMesh(core_axis_name="c", subcore_axis_name="s")

_cp = pltpu.CompilerParams()
if "needs_layout_passes" in pltpu.CompilerParams.__dataclass_fields__:
    _cp = dataclasses.replace(_cp, needs_layout_passes=False)


@functools.partial(
    pl.kernel,
    compiler_params=_cp,
    out_type=jax.ShapeDtypeStruct((NC, A_ROWS, D), jnp.float32),
    mesh=_mesh,
    scratch_types=[
        pltpu.VMEM((CHUNK, D), jnp.float32),
        pltpu.VMEM((CHUNK, D), jnp.float32),
        pltpu.VMEM((H_ROWS, D), jnp.float32),
        pltpu.VMEM((SUP, CHUNK), jnp.int32),
        pltpu.VMEM((SUP, CHUNK), jnp.int32),
        pltpu.VMEM((H_ROWS,), jnp.int32),
        pltpu.VMEM_SHARED((A_ROWS, D), jnp.float32),
        pltpu.SemaphoreType.DMA,
        pltpu.SemaphoreType.DMA,
        pltpu.SemaphoreType.DMA,
        pltpu.SemaphoreType.DMA,
    ],
)
def _sc_aggregate(src_hbm, dst_hbm, h_hbm, p_out,
                  rows_v, rows_b, hist_v, src_v, dst_v, fold_v, acc_s,
                  sem_ga, sem_gb, sem_sa, sem_sb):
    c = lax.axis_index("c")
    s = lax.axis_index("s")

    @pl.loop(0, CHUNK)
    def _zero_rows(i):
        @pl.loop(0, D, step=L)
        def _zero_cols(jc):
            rows_v[i, pl.ds(jc, L)] = jnp.zeros((L,), jnp.float32)

    @pl.loop(0, H_ROWS)
    def _zero_hist(i):
        @pl.loop(0, D, step=L)
        def _zero_hcols(jc):
            hist_v[i, pl.ds(jc, L)] = jnp.zeros((L,), jnp.float32)
    base_row = s * ROWS_PER_SUB
    for off_k, len_k in ((0, 128), (128, 128), (256, 128), (384, 128),
                         (512, 128), (640, 8)):
        pltpu.sync_copy(rows_v.at[pl.ds(0, len_k)],
                        acc_s.at[pl.ds(base_row + off_k, len_k)])

    plsc.subcore_barrier()

    wid = c * NS + s
    lane = lax.iota(jnp.int32, L)
    ones16 = jnp.ones((L,), jnp.float32)

    def _hist_update(jj):
        for m in range(CHUNK // L):
            d16 = dst_v[jj, pl.ds(m * L, L)]
            r16 = lax.shift_right_logical(d16, 7)
            c16 = jnp.bitwise_and(d16, 127)
            plsc.addupdate_scatter(hist_v, [r16, c16], ones16)

    @pl.loop(0, J // SUP)
    def _super_block(sup):
        row0 = wid * J + sup * SUP
        pltpu.sync_copy(src_hbm.at[pl.ds(row0, SUP)], src_v)
        pltpu.sync_copy(dst_hbm.at[pl.ds(row0, SUP)], dst_v)
        pltpu.async_copy(h_hbm.at[src_v.at[0]], rows_v, sem_ga)

        @pl.loop(0, SUP, step=2)
        def _pair(jj):
            pltpu.make_async_copy(h_hbm.at[src_v.at[jj]], rows_v, sem_ga).wait()
            pltpu.async_copy(h_hbm.at[src_v.at[jj + 1]], rows_b, sem_gb)
            pltpu.async_copy(rows_v, acc_s.at[dst_v.at[jj]], sem_sa, add=True)
            _hist_update(jj)
            pltpu.make_async_copy(h_hbm.at[src_v.at[jj + 1]], rows_b,
                                  sem_gb).wait()
            pltpu.make_async_copy(rows_v, acc_s.at[dst_v.at[jj]],
                                  sem_sa).wait()

            @pl.when(jj < SUP - 2)
            def _prefetch():
                pltpu.async_copy(h_hbm.at[src_v.at[jj + 2]], rows_v, sem_ga)

            pltpu.async_copy(rows_b, acc_s.at[dst_v.at[jj + 1]], sem_sb,
                             add=True)
            _hist_update(jj + 1)
            pltpu.make_async_copy(rows_b, acc_s.at[dst_v.at[jj + 1]],
                                  sem_sb).wait()

    @pl.loop(0, H_ROWS, step=L)
    def _fill_fold(i):
        fold_v[pl.ds(i, L)] = lane + (N_PAD + i)

    pltpu.sync_copy(hist_v, acc_s.at[fold_v], add=True)

    plsc.subcore_barrier()

    pltpu.sync_copy(acc_s.at[pl.ds(base_row, ROWS_PER_SUB)],
                    p_out.at[c].at[pl.ds(base_row, ROWS_PER_SUB)])


_BLK = 2000


def _tc_body(p_ref, d_ref, w_ref, b_ref, o_ref):
    p = p_ref[0] + p_ref[1]
    pn = p / jnp.maximum(d_ref[...], 1.0)
    acc = lax.dot_general(pn, w_ref[...], (((1,), (1,)), ((), ())),
                          preferred_element_type=jnp.float32)
    o_ref[...] = acc + b_ref[...]


def _tc_finish(p_parts, degcol, W, b2):
    return pl.pallas_call(
        _tc_body,
        grid=(N_NODES // _BLK,),
        in_specs=[
            pl.BlockSpec((NC, _BLK, D), lambda i: (0, i, 0)),
            pl.BlockSpec((_BLK, 1), lambda i: (i, 0)),
            pl.BlockSpec((D, D), lambda i: (0, 0)),
            pl.BlockSpec((1, D), lambda i: (0, 0)),
        ],
        out_specs=pl.BlockSpec((_BLK, D), lambda i: (i, 0)),
        out_shape=jax.ShapeDtypeStruct((N_NODES, D), jnp.float32),
    )(p_parts, degcol, W, b2)


def kernel(h, edge_index, W, b):
    src = edge_index[0].astype(jnp.int32)
    dst = edge_index[1].astype(jnp.int32)
    pad = E_PAD - N_EDGES
    pad_iota = jnp.arange(pad, dtype=jnp.int32)
    src_p = jnp.concatenate([src, pad_iota % N_NODES]).reshape(-1, CHUNK)
    dst_p = jnp.concatenate(
        [dst, N_NODES + (pad_iota % (N_PAD - N_NODES))]).reshape(-1, CHUNK)
    p_parts = _sc_aggregate(src_p, dst_p, h)
    deg = p_parts[0, N_PAD:N_PAD + H_ROWS] + p_parts[1, N_PAD:N_PAD + H_ROWS]
    degcol = deg.reshape(N_PAD)[:N_NODES, None]
    return _tc_finish(p_parts, degcol, W, b.reshape(1, D))


# speedup vs baseline: 12.7218x; 1.0499x over previous
import dataclasses
import functools

import jax
import jax.numpy as jnp
from jax import lax
from jax.experimental import pallas as pl
from jax.experimental.pallas import tpu as pltpu
from jax.experimental.pallas import tpu_sc as plsc

N_NODES = 10000
N_EDGES = 320000
D = 128
NC = 2
NS = 16
L = 16
CHUNK = 128
J = 80
SUP = 8
E_PAD = NC * NS * J * CHUNK
N_PAD = 10240
H_ROWS = N_PAD // D
A_ROWS = 10368
ROWS_PER_SUB = A_ROWS // NS

_mesh = plsc.VectorSubcoreMesh(core_axis_name="c", subcore_axis_name="s")

_cp = pltpu.CompilerParams()
if "needs_layout_passes" in pltpu.CompilerParams.__dataclass_fields__:
    _cp = dataclasses.replace(_cp, needs_layout_passes=False)


@functools.partial(
    pl.kernel,
    compiler_params=_cp,
    out_type=jax.ShapeDtypeStruct((NC, A_ROWS, D), jnp.float32),
    mesh=_mesh,
    scratch_types=[
        pltpu.VMEM((CHUNK, D), jnp.float32),
        pltpu.VMEM((CHUNK, D), jnp.float32),
        pltpu.VMEM((H_ROWS, D), jnp.float32),
        pltpu.VMEM((SUP, CHUNK), jnp.int32),
        pltpu.VMEM((SUP, CHUNK), jnp.int32),
        pltpu.VMEM((SUP, CHUNK), jnp.int32),
        pltpu.VMEM((SUP, CHUNK), jnp.int32),
        pltpu.VMEM((H_ROWS,), jnp.int32),
        pltpu.VMEM_SHARED((A_ROWS, D), jnp.float32),
        pltpu.SemaphoreType.DMA,
        pltpu.SemaphoreType.DMA,
        pltpu.SemaphoreType.DMA,
        pltpu.SemaphoreType.DMA,
        pltpu.SemaphoreType.DMA,
        pltpu.SemaphoreType.DMA,
    ],
)
def _sc_aggregate(src_hbm, dst_hbm, h_hbm, p_out,
                  rows_v, rows_b, hist_v, src_a, dst_a, src_b, dst_b,
                  fold_v, acc_s,
                  sem_ga, sem_gb, sem_sa, sem_sb, sem_ia, sem_ib):
    c = lax.axis_index("c")
    s = lax.axis_index("s")

    @pl.loop(0, CHUNK)
    def _zero_rows(i):
        @pl.loop(0, D, step=L)
        def _zero_cols(jc):
            rows_v[i, pl.ds(jc, L)] = jnp.zeros((L,), jnp.float32)

    @pl.loop(0, H_ROWS)
    def _zero_hist(i):
        @pl.loop(0, D, step=L)
        def _zero_hcols(jc):
            hist_v[i, pl.ds(jc, L)] = jnp.zeros((L,), jnp.float32)
    base_row = s * ROWS_PER_SUB
    for off_k, len_k in ((0, 128), (128, 128), (256, 128), (384, 128),
                         (512, 128), (640, 8)):
        pltpu.sync_copy(rows_v.at[pl.ds(0, len_k)],
                        acc_s.at[pl.ds(base_row + off_k, len_k)])

    plsc.subcore_barrier()

    wid = c * NS + s
    lane = lax.iota(jnp.int32, L)
    ones16 = jnp.ones((L,), jnp.float32)

    def _hist_update(jj, dstb):
        for m in range(CHUNK // L):
            d16 = dstb[jj, pl.ds(m * L, L)]
            r16 = lax.shift_right_logical(d16, 7)
            c16 = jnp.bitwise_and(d16, 127)
            plsc.addupdate_scatter(hist_v, [r16, c16], ones16)

    def _pair_loop(srcb, dstb):
        @pl.loop(0, SUP, step=2)
        def _pair(jj):
            pltpu.make_async_copy(h_hbm.at[srcb.at[jj]], rows_v, sem_ga).wait()

            @pl.when(jj > 0)
            def _drain_sb():
                pltpu.make_async_copy(rows_b, acc_s.at[dstb.at[jj]],
                                      sem_sb).wait()

            pltpu.async_copy(h_hbm.at[srcb.at[jj + 1]], rows_b, sem_gb)
            pltpu.async_copy(rows_v, acc_s.at[dstb.at[jj]], sem_sa, add=True)
            _hist_update(jj, dstb)
            pltpu.make_async_copy(h_hbm.at[srcb.at[jj + 1]], rows_b,
                                  sem_gb).wait()
            pltpu.make_async_copy(rows_v, acc_s.at[dstb.at[jj]],
                                  sem_sa).wait()

            @pl.when(jj < SUP - 2)
            def _prefetch():
                pltpu.async_copy(h_hbm.at[srcb.at[jj + 2]], rows_v, sem_ga)

            pltpu.async_copy(rows_b, acc_s.at[dstb.at[jj + 1]], sem_sb,
                             add=True)
            _hist_update(jj + 1, dstb)

        pltpu.make_async_copy(rows_b, acc_s.at[dstb.at[0]], sem_sb).wait()

    nsup = J // SUP
    base0 = wid * J
    pltpu.async_copy(src_hbm.at[pl.ds(base0, SUP)], src_a, sem_ia)
    pltpu.async_copy(dst_hbm.at[pl.ds(base0, SUP)], dst_a, sem_ia)

    @pl.loop(0, nsup // 2)
    def _super_pair(sp):
        base_a = wid * J + sp * 2 * SUP
        base_b = base_a + SUP
        pltpu.make_async_copy(src_hbm.at[pl.ds(base_a, SUP)], src_a,
                              sem_ia).wait()
        pltpu.make_async_copy(dst_hbm.at[pl.ds(base_a, SUP)], dst_a,
                              sem_ia).wait()
        pltpu.async_copy(src_hbm.at[pl.ds(base_b, SUP)], src_b, sem_ib)
        pltpu.async_copy(dst_hbm.at[pl.ds(base_b, SUP)], dst_b, sem_ib)
        pltpu.async_copy(h_hbm.at[src_a.at[0]], rows_v, sem_ga)
        _pair_loop(src_a, dst_a)
        pltpu.make_async_copy(src_hbm.at[pl.ds(base_b, SUP)], src_b,
                              sem_ib).wait()
        pltpu.make_async_copy(dst_hbm.at[pl.ds(base_b, SUP)], dst_b,
                              sem_ib).wait()

        @pl.when(sp < nsup // 2 - 1)
        def _prefetch_idx():
            nbase = base_a + 2 * SUP
            pltpu.async_copy(src_hbm.at[pl.ds(nbase, SUP)], src_a, sem_ia)
            pltpu.async_copy(dst_hbm.at[pl.ds(nbase, SUP)], dst_a, sem_ia)

        pltpu.async_copy(h_hbm.at[src_b.at[0]], rows_v, sem_ga)
        _pair_loop(src_b, dst_b)

    @pl.loop(0, H_ROWS, step=L)
    def _fill_fold(i):
        fold_v[pl.ds(i, L)] = lane + (N_PAD + i)

    pltpu.sync_copy(hist_v, acc_s.at[fold_v], add=True)

    plsc.subcore_barrier()

    pltpu.sync_copy(acc_s.at[pl.ds(base_row, ROWS_PER_SUB)],
                    p_out.at[c].at[pl.ds(base_row, ROWS_PER_SUB)])


_BLK = 2000


def _tc_body(p_ref, d_ref, w_ref, b_ref, o_ref):
    p = p_ref[0] + p_ref[1]
    pn = p / jnp.maximum(d_ref[...], 1.0)
    acc = lax.dot_general(pn, w_ref[...], (((1,), (1,)), ((), ())),
                          preferred_element_type=jnp.float32)
    o_ref[...] = acc + b_ref[...]


def _tc_finish(p_parts, degcol, W, b2):
    return pl.pallas_call(
        _tc_body,
        grid=(N_NODES // _BLK,),
        in_specs=[
            pl.BlockSpec((NC, _BLK, D), lambda i: (0, i, 0)),
            pl.BlockSpec((_BLK, 1), lambda i: (i, 0)),
            pl.BlockSpec((D, D), lambda i: (0, 0)),
            pl.BlockSpec((1, D), lambda i: (0, 0)),
        ],
        out_specs=pl.BlockSpec((_BLK, D), lambda i: (i, 0)),
        out_shape=jax.ShapeDtypeStruct((N_NODES, D), jnp.float32),
    )(p_parts, degcol, W, b2)


def kernel(h, edge_index, W, b):
    src = edge_index[0].astype(jnp.int32)
    dst = edge_index[1].astype(jnp.int32)
    pad = E_PAD - N_EDGES
    pad_iota = jnp.arange(pad, dtype=jnp.int32)
    src_p = jnp.concatenate([src, pad_iota % N_NODES]).reshape(-1, CHUNK)
    dst_p = jnp.concatenate(
        [dst, N_NODES + (pad_iota % (N_PAD - N_NODES))]).reshape(-1, CHUNK)
    p_parts = _sc_aggregate(src_p, dst_p, h)
    deg = p_parts[0, N_PAD:N_PAD + H_ROWS] + p_parts[1, N_PAD:N_PAD + H_ROWS]
    degcol = deg.reshape(N_PAD)[:N_NODES, None]
    return _tc_finish(p_parts, degcol, W, b.reshape(1, D))


# speedup vs baseline: 12.7953x; 1.0058x over previous
import dataclasses
import functools

import jax
import jax.numpy as jnp
from jax import lax
from jax.experimental import pallas as pl
from jax.experimental.pallas import tpu as pltpu
from jax.experimental.pallas import tpu_sc as plsc

N_NODES = 10000
N_EDGES = 320000
D = 128
NC = 2
NS = 16
L = 16
CHUNK = 128
J = 80
SUP = 8
E_PAD = NC * NS * J * CHUNK
N_PAD = 10080
H0 = 10080
H_ROWS = 80
A_ROWS = 10240
ROWS_PER_SUB = A_ROWS // NS

_mesh = plsc.VectorSubcoreMesh(core_axis_name="c", subcore_axis_name="s")

_cp = pltpu.CompilerParams()
if "needs_layout_passes" in pltpu.CompilerParams.__dataclass_fields__:
    _cp = dataclasses.replace(_cp, needs_layout_passes=False)


@functools.partial(
    pl.kernel,
    compiler_params=_cp,
    out_type=jax.ShapeDtypeStruct((NC, A_ROWS, D), jnp.float32),
    mesh=_mesh,
    scratch_types=[
        pltpu.VMEM((CHUNK, D), jnp.float32),
        pltpu.VMEM((CHUNK, D), jnp.float32),
        pltpu.VMEM((H_ROWS, D), jnp.float32),
        pltpu.VMEM((SUP, CHUNK), jnp.int32),
        pltpu.VMEM((SUP, CHUNK), jnp.int32),
        pltpu.VMEM((SUP, CHUNK), jnp.int32),
        pltpu.VMEM((SUP, CHUNK), jnp.int32),
        pltpu.VMEM((H_ROWS,), jnp.int32),
        pltpu.VMEM_SHARED((A_ROWS, D), jnp.float32),
        pltpu.SemaphoreType.DMA,
        pltpu.SemaphoreType.DMA,
        pltpu.SemaphoreType.DMA,
        pltpu.SemaphoreType.DMA,
        pltpu.SemaphoreType.DMA,
        pltpu.SemaphoreType.DMA,
    ],
)
def _sc_aggregate(src_hbm, dst_hbm, h_hbm, p_out,
                  rows_v, rows_b, hist_v, src_a, dst_a, src_b, dst_b,
                  fold_v, acc_s,
                  sem_ga, sem_gb, sem_sa, sem_sb, sem_ia, sem_ib):
    c = lax.axis_index("c")
    s = lax.axis_index("s")

    @pl.loop(0, CHUNK)
    def _zero_rows(i):
        @pl.loop(0, D, step=L)
        def _zero_cols(jc):
            rows_v[i, pl.ds(jc, L)] = jnp.zeros((L,), jnp.float32)

    @pl.loop(0, H_ROWS)
    def _zero_hist(i):
        @pl.loop(0, D, step=L)
        def _zero_hcols(jc):
            hist_v[i, pl.ds(jc, L)] = jnp.zeros((L,), jnp.float32)
    base_row = s * ROWS_PER_SUB
    for k in range(ROWS_PER_SUB // CHUNK):
        pltpu.sync_copy(rows_v, acc_s.at[pl.ds(base_row + k * CHUNK, CHUNK)])

    plsc.subcore_barrier()

    wid = c * NS + s
    lane = lax.iota(jnp.int32, L)
    ones16 = jnp.ones((L,), jnp.float32)

    def _hist_update(jj, dstb):
        for m in range(CHUNK // L):
            d16 = dstb[jj, pl.ds(m * L, L)]
            r16 = lax.shift_right_logical(d16, 7)
            c16 = jnp.bitwise_and(d16, 127)
            plsc.addupdate_scatter(hist_v, [r16, c16], ones16)

    def _pair_loop(srcb, dstb):
        @pl.loop(0, SUP, step=2)
        def _pair(jj):
            pltpu.make_async_copy(h_hbm.at[srcb.at[jj]], rows_v, sem_ga).wait()

            @pl.when(jj > 0)
            def _drain_sb():
                pltpu.make_async_copy(rows_b, acc_s.at[dstb.at[jj]],
                                      sem_sb).wait()

            pltpu.async_copy(h_hbm.at[srcb.at[jj + 1]], rows_b, sem_gb)
            pltpu.async_copy(rows_v, acc_s.at[dstb.at[jj]], sem_sa, add=True)
            _hist_update(jj, dstb)
            pltpu.make_async_copy(h_hbm.at[srcb.at[jj + 1]], rows_b,
                                  sem_gb).wait()
            pltpu.make_async_copy(rows_v, acc_s.at[dstb.at[jj]],
                                  sem_sa).wait()

            @pl.when(jj < SUP - 2)
            def _prefetch():
                pltpu.async_copy(h_hbm.at[srcb.at[jj + 2]], rows_v, sem_ga)

            pltpu.async_copy(rows_b, acc_s.at[dstb.at[jj + 1]], sem_sb,
                             add=True)
            _hist_update(jj + 1, dstb)

        pltpu.make_async_copy(rows_b, acc_s.at[dstb.at[0]], sem_sb).wait()

    nsup = J // SUP
    base0 = wid * J
    pltpu.async_copy(src_hbm.at[pl.ds(base0, SUP)], src_a, sem_ia)
    pltpu.async_copy(dst_hbm.at[pl.ds(base0, SUP)], dst_a, sem_ia)

    @pl.loop(0, nsup // 2)
    def _super_pair(sp):
        base_a = wid * J + sp * 2 * SUP
        base_b = base_a + SUP
        pltpu.make_async_copy(src_hbm.at[pl.ds(base_a, SUP)], src_a,
                              sem_ia).wait()
        pltpu.make_async_copy(dst_hbm.at[pl.ds(base_a, SUP)], dst_a,
                              sem_ia).wait()
        pltpu.async_copy(src_hbm.at[pl.ds(base_b, SUP)], src_b, sem_ib)
        pltpu.async_copy(dst_hbm.at[pl.ds(base_b, SUP)], dst_b, sem_ib)
        pltpu.async_copy(h_hbm.at[src_a.at[0]], rows_v, sem_ga)
        _pair_loop(src_a, dst_a)
        pltpu.make_async_copy(src_hbm.at[pl.ds(base_b, SUP)], src_b,
                              sem_ib).wait()
        pltpu.make_async_copy(dst_hbm.at[pl.ds(base_b, SUP)], dst_b,
                              sem_ib).wait()

        @pl.when(sp < nsup // 2 - 1)
        def _prefetch_idx():
            nbase = base_a + 2 * SUP
            pltpu.async_copy(src_hbm.at[pl.ds(nbase, SUP)], src_a, sem_ia)
            pltpu.async_copy(dst_hbm.at[pl.ds(nbase, SUP)], dst_a, sem_ia)

        pltpu.async_copy(h_hbm.at[src_b.at[0]], rows_v, sem_ga)
        _pair_loop(src_b, dst_b)

    @pl.loop(0, H_ROWS, step=L)
    def _fill_fold(i):
        fold_v[pl.ds(i, L)] = lane + (H0 + i)

    pltpu.sync_copy(hist_v, acc_s.at[fold_v], add=True)

    plsc.subcore_barrier()

    pltpu.sync_copy(acc_s.at[pl.ds(base_row, ROWS_PER_SUB)],
                    p_out.at[c].at[pl.ds(base_row, ROWS_PER_SUB)])


_BLK = 2000


def _tc_body(p_ref, d_ref, w_ref, b_ref, o_ref):
    p = p_ref[0] + p_ref[1]
    pn = p / jnp.maximum(d_ref[...], 1.0)
    acc = lax.dot_general(pn, w_ref[...], (((1,), (1,)), ((), ())),
                          preferred_element_type=jnp.float32)
    o_ref[...] = acc + b_ref[...]


def _tc_finish(p_parts, degcol, W, b2):
    return pl.pallas_call(
        _tc_body,
        grid=(N_NODES // _BLK,),
        in_specs=[
            pl.BlockSpec((NC, _BLK, D), lambda i: (0, i, 0)),
            pl.BlockSpec((_BLK, 1), lambda i: (i, 0)),
            pl.BlockSpec((D, D), lambda i: (0, 0)),
            pl.BlockSpec((1, D), lambda i: (0, 0)),
        ],
        out_specs=pl.BlockSpec((_BLK, D), lambda i: (i, 0)),
        out_shape=jax.ShapeDtypeStruct((N_NODES, D), jnp.float32),
    )(p_parts, degcol, W, b2)


def kernel(h, edge_index, W, b):
    src = edge_index[0].astype(jnp.int32)
    dst = edge_index[1].astype(jnp.int32)
    pad = E_PAD - N_EDGES
    pad_iota = jnp.arange(pad, dtype=jnp.int32)
    src_p = jnp.concatenate([src, pad_iota % N_NODES]).reshape(-1, CHUNK)
    dst_p = jnp.concatenate(
        [dst, N_NODES + (pad_iota % (N_PAD - N_NODES))]).reshape(-1, CHUNK)
    p_parts = _sc_aggregate(src_p, dst_p, h)
    deg = p_parts[0, H0:H0 + H_ROWS] + p_parts[1, H0:H0 + H_ROWS]
    degcol = deg.reshape(H_ROWS * D)[:N_NODES, None]
    return _tc_finish(p_parts, degcol, W, b.reshape(1, D))
